# baseline (speedup 1.0000x reference)
import functools

import jax
import jax.numpy as jnp
from jax.experimental import pallas as pl
from jax.experimental.pallas import tpu as pltpu

DIM = 384
MLP_DIM = 768
HEADS = 8
TOP_K = 256
DH = DIM // HEADS

B = 4
LQ = 1024
LK = 4096
NQ = 256


def _f32dot(a, b, precision=jax.lax.Precision.HIGHEST):
    return jax.lax.dot_general(a, b, (((1,), (0,)), ((), ())),
                               preferred_element_type=jnp.float32,
                               precision=precision)


def _exactdot(a, b):
    return _f32dot(a, b, precision=jax.lax.Precision.HIGHEST)


def _dist_kernel(kv_ref, q_ref, roh_ref, mind_ref):
    qs = _exactdot(roh_ref[0], q_ref[0])

    def tree128(p):
        s = 64
        while s >= 1:
            p = p[..., 0:s] + p[..., s:2 * s]
            s //= 2
        return p[..., 0]

    def body(c, _):
        k16 = kv_ref[0, pl.ds(c * 16, 16), :]
        dd = jnp.abs(k16[:, None, :] - qs[None, :, :])
        s = (tree128(dd[..., 0:128]) + tree128(dd[..., 128:256])) \
            + tree128(dd[..., 256:384])
        mind_ref[0, c, :] = s.min(axis=1)
        return _

    jax.lax.fori_loop(0, LK // 16, body, 0)


def _attn_kernel(mind_ref, kv_ref, q_ref, wi_ref, bi_ref, wo_ref, bo_ref,
                 out_ref, rank_s, mask_s):
    mi = jax.lax.bitcast_convert_type(mind_ref[0], jnp.int32)

    def bit_body(i, t):
        b = 30 - i
        cand = t | ((1 << b) - 1)
        cnt = jnp.sum((mi <= cand).astype(jnp.int32))
        return jnp.where(cnt >= TOP_K, t, t | (1 << b))

    thr = jax.lax.fori_loop(0, 31, bit_body, jnp.int32(0))

    maskf = (mi <= thr).astype(jnp.float32)
    lane_i = jax.lax.broadcasted_iota(jnp.int32, (128, 128), 0)
    lane_j = jax.lax.broadcasted_iota(jnp.int32, (128, 128), 1)
    upper = (lane_i <= lane_j).astype(jnp.float32)
    incl = _f32dot(maskf, upper)
    rowtot = incl[:, 127:128]
    row_i = jax.lax.broadcasted_iota(jnp.int32, (32, 32), 0)
    row_j = jax.lax.broadcasted_iota(jnp.int32, (32, 32), 1)
    strict = (row_j < row_i).astype(jnp.float32)
    offs = _f32dot(strict, rowtot)
    rank_s[...] = offs + incl - maskf
    mask_s[...] = maskf

    slot = jax.lax.broadcasted_iota(jnp.int32, (TOP_K, 128), 0).astype(
        jnp.float32)

    def gather_body(c, sel):
        rrow = rank_s[pl.ds(c, 1), :]
        mrow = mask_s[pl.ds(c, 1), :]
        oh = jnp.where((slot == rrow) & (mrow > 0.5), 1.0, 0.0)
        kvc = kv_ref[0, pl.ds(c * 128, 128), :]
        return sel + _exactdot(oh, kvc)

    sel = jax.lax.fori_loop(0, 32, gather_body,
                            jnp.zeros((TOP_K, DIM), jnp.float32))

    wi = wi_ref[...]
    bi = bi_ref[...]
    q = q_ref[0]
    qp = _f32dot(q, wi[0:DIM, :].T) + bi[:, 0:DIM]
    kp = _f32dot(sel, wi[DIM:2 * DIM, :].T) + bi[:, DIM:2 * DIM]
    vp = _f32dot(sel, wi[2 * DIM:3 * DIM, :].T) + bi[:, 2 * DIM:3 * DIM]

    scale = 1.0 / jnp.sqrt(jnp.float32(DH))
    outs = []
    for h in range(HEADS):
        lo, hi = h * DH, (h + 1) * DH
        s = _f32dot(qp[:, lo:hi], kp[:, lo:hi].T) * scale
        s = s - s.max(axis=1, keepdims=True)
        e = jnp.exp(s)
        p = e / e.sum(axis=1, keepdims=True)
        outs.append(_f32dot(p, vp[:, lo:hi]))
    o = jnp.concatenate(outs, axis=1)
    out_ref[0] = _f32dot(o, wo_ref[...].T) + bo_ref[...]


def _mlp_kernel(x_ref, w1_ref, b1_ref, g1_ref, be1_ref,
                w2_ref, b2_ref, g2_ref, be2_ref, out_ref):
    x = x_ref[...]
    h1 = _f32dot(x, w1_ref[...].T) + b1_ref[...]
    m1 = jnp.mean(h1, axis=0, keepdims=True)
    d1 = h1 - m1
    v1 = jnp.mean(d1 * d1, axis=0, keepdims=True)
    h1n = d1 / jnp.sqrt(v1 + 1e-5) * g1_ref[...] + be1_ref[...]
    h1n = jnp.maximum(h1n, 0.0)
    h2 = _f32dot(h1n, w2_ref[...].T) + b2_ref[...]
    m2 = jnp.mean(h2, axis=0, keepdims=True)
    d2 = h2 - m2
    v2 = jnp.mean(d2 * d2, axis=0, keepdims=True)
    h2n = d2 / jnp.sqrt(v2 + 1e-5) * g2_ref[...] + be2_ref[...]
    out_ref[...] = jnp.maximum(h2n + x, 0.0)


@functools.partial(jax.jit, static_argnums=())
def kernel(query_source, context, in_proj_w, in_proj_b, out_proj_w, out_proj_b,
           conv1_w, conv1_b, bn1_g, bn1_b, conv2_w, conv2_b, bn2_g, bn2_b):
    b, c, hq, wq = query_source.shape
    q = query_source.reshape(b, c, -1).transpose(0, 2, 1)
    kv = context.reshape(b, c, -1).transpose(0, 2, 1)

    rand_ind = jax.random.randint(jax.random.key(42), (b, NQ), 0, q.shape[1])
    roh = jax.nn.one_hot(rand_ind, q.shape[1], dtype=jnp.float32)

    mind = pl.pallas_call(
        _dist_kernel,
        grid=(b,),
        in_specs=[pl.BlockSpec((1, LK, DIM), lambda i: (i, 0, 0)),
                  pl.BlockSpec((1, LQ, DIM), lambda i: (i, 0, 0)),
                  pl.BlockSpec((1, NQ, LQ), lambda i: (i, 0, 0))],
        out_specs=pl.BlockSpec((1, LK // 16, 16), lambda i: (i, 0, 0)),
        out_shape=jax.ShapeDtypeStruct((b, LK // 16, 16), jnp.float32),
    )(kv, q, roh)
    mind_r = mind.reshape(b, 32, 128)

    attn = pl.pallas_call(
        _attn_kernel,
        grid=(b,),
        in_specs=[pl.BlockSpec((1, 32, 128), lambda i: (i, 0, 0)),
                  pl.BlockSpec((1, LK, DIM), lambda i: (i, 0, 0)),
                  pl.BlockSpec((1, LQ, DIM), lambda i: (i, 0, 0)),
                  pl.BlockSpec((3 * DIM, DIM), lambda i: (0, 0)),
                  pl.BlockSpec((1, 3 * DIM), lambda i: (0, 0)),
                  pl.BlockSpec((DIM, DIM), lambda i: (0, 0)),
                  pl.BlockSpec((1, DIM), lambda i: (0, 0))],
        out_specs=pl.BlockSpec((1, LQ, DIM), lambda i: (i, 0, 0)),
        out_shape=jax.ShapeDtypeStruct((b, LQ, DIM), jnp.float32),
        scratch_shapes=[pltpu.VMEM((32, 128), jnp.float32),
                        pltpu.VMEM((32, 128), jnp.float32)],
        compiler_params=pltpu.CompilerParams(vmem_limit_bytes=110 * 2**20),
    )(mind_r, kv, q, in_proj_w, in_proj_b.reshape(1, -1),
      out_proj_w, out_proj_b.reshape(1, -1))

    x = attn.reshape(b * LQ, DIM)
    out = pl.pallas_call(
        _mlp_kernel,
        in_specs=[pl.BlockSpec((b * LQ, DIM), lambda: (0, 0)),
                  pl.BlockSpec((MLP_DIM, DIM), lambda: (0, 0)),
                  pl.BlockSpec((1, MLP_DIM), lambda: (0, 0)),
                  pl.BlockSpec((1, MLP_DIM), lambda: (0, 0)),
                  pl.BlockSpec((1, MLP_DIM), lambda: (0, 0)),
                  pl.BlockSpec((DIM, MLP_DIM), lambda: (0, 0)),
                  pl.BlockSpec((1, DIM), lambda: (0, 0)),
                  pl.BlockSpec((1, DIM), lambda: (0, 0)),
                  pl.BlockSpec((1, DIM), lambda: (0, 0))],
        out_specs=pl.BlockSpec((b * LQ, DIM), lambda: (0, 0)),
        out_shape=jax.ShapeDtypeStruct((b * LQ, DIM), jnp.float32),
        compiler_params=pltpu.CompilerParams(vmem_limit_bytes=110 * 2**20),
    )(x, conv1_w, conv1_b.reshape(1, -1), bn1_g.reshape(1, -1),
      bn1_b.reshape(1, -1), conv2_w, conv2_b.reshape(1, -1),
      bn2_g.reshape(1, -1), bn2_b.reshape(1, -1))

    return out.reshape(b, LQ, c).transpose(0, 2, 1).reshape(b, c, hq, wq)
